# Initial kernel scaffold; baseline (speedup 1.0000x reference)
import jax
import jax.numpy as jnp
from jax.experimental import pallas as pl


def kernel(hidden_states, router_weight, router_bias, gate_up_proj, gate_up_proj_bias, down_proj, down_proj_bias):
    raise NotImplementedError("write your pallas kernel here")


# speedup vs baseline: 1.0084x; 1.0084x over previous
import functools

import jax
import jax.numpy as jnp
from jax.experimental import pallas as pl
from jax.experimental.pallas import tpu as pltpu

E = 8
D = 768
FF = 768
ALPHA = 1.702
LIMIT = 7.0


def _moe_body(hs_ref, rwt_ref, rb_ref, gw_ref, gb_ref, uw_ref, ub_ref,
              dw_ref, db_ref, out_ref, scores_ref, *, bt):
    e = pl.program_id(0)
    t = pl.program_id(1)
    x = hs_ref[...]

    @pl.when(e == 0)
    def _router():
        logits = jnp.dot(x, rwt_ref[...], preferred_element_type=jnp.float32)
        logits = logits + rb_ref[...]
        col = jax.lax.broadcasted_iota(jnp.int32, logits.shape, 1)
        m1 = jnp.max(logits, axis=1, keepdims=True)
        a1 = jnp.min(jnp.where(logits == m1, col, E), axis=1, keepdims=True)
        rest = jnp.where(col == a1, -jnp.inf, logits)
        m2 = jnp.max(rest, axis=1, keepdims=True)
        a2 = jnp.min(jnp.where(rest == m2, col, E), axis=1, keepdims=True)
        p1 = 1.0 / (1.0 + jnp.exp(m2 - m1))
        p2 = 1.0 - p1
        scores = jnp.where(col == a1, p1, jnp.where(col == a2, p2, 0.0))
        scores_ref[pl.ds(t * bt, bt), :] = scores

    scores_blk = scores_ref[pl.ds(t * bt, bt), :]
    col = jax.lax.broadcasted_iota(jnp.int32, scores_blk.shape, 1)
    w = jnp.sum(jnp.where(col == e, scores_blk, 0.0), axis=1, keepdims=True)

    g = jnp.dot(x, gw_ref[0], preferred_element_type=jnp.float32) + gb_ref[0]
    u = jnp.dot(x, uw_ref[0], preferred_element_type=jnp.float32) + ub_ref[0]
    g = jnp.minimum(g, LIMIT)
    u = jnp.clip(u, -LIMIT, LIMIT)
    glu = g / (1.0 + jnp.exp(-ALPHA * g))
    act = (u + 1.0) * glu
    contrib = jnp.dot(act, dw_ref[0], preferred_element_type=jnp.float32)
    contrib = w * (contrib + db_ref[0])

    sl = pl.ds(t * bt, bt)

    @pl.when(e == 0)
    def _init():
        out_ref[sl, :] = contrib

    @pl.when(e != 0)
    def _acc():
        out_ref[sl, :] = out_ref[sl, :] + contrib


def kernel(hidden_states, router_weight, router_bias, gate_up_proj,
           gate_up_proj_bias, down_proj, down_proj_bias):
    bsz, seq, d = hidden_states.shape
    T = bsz * seq
    hs = hidden_states.reshape(T, d)
    BT = 512
    NT = T // BT

    rwt = router_weight.T
    rb = router_bias.reshape(1, E)
    gate_w = gate_up_proj[..., ::2]
    up_w = gate_up_proj[..., 1::2]
    gate_b = gate_up_proj_bias[..., ::2].reshape(E, 1, FF)
    up_b = gate_up_proj_bias[..., 1::2].reshape(E, 1, FF)
    down_b = down_proj_bias.reshape(E, 1, D)

    grid = (E, NT)
    out, scores = pl.pallas_call(
        functools.partial(_moe_body, bt=BT),
        grid=grid,
        in_specs=[
            pl.BlockSpec((BT, D), lambda e, t: (t, 0)),
            pl.BlockSpec((D, E), lambda e, t: (0, 0)),
            pl.BlockSpec((1, E), lambda e, t: (0, 0)),
            pl.BlockSpec((1, D, FF), lambda e, t: (e, 0, 0)),
            pl.BlockSpec((1, 1, FF), lambda e, t: (e, 0, 0)),
            pl.BlockSpec((1, D, FF), lambda e, t: (e, 0, 0)),
            pl.BlockSpec((1, 1, FF), lambda e, t: (e, 0, 0)),
            pl.BlockSpec((1, FF, D), lambda e, t: (e, 0, 0)),
            pl.BlockSpec((1, 1, D), lambda e, t: (e, 0, 0)),
        ],
        out_specs=[
            pl.BlockSpec((T, D), lambda e, t: (0, 0)),
            pl.BlockSpec((T, E), lambda e, t: (0, 0)),
        ],
        out_shape=[
            jax.ShapeDtypeStruct((T, D), jnp.float32),
            jax.ShapeDtypeStruct((T, E), jnp.float32),
        ],
        compiler_params=pltpu.CompilerParams(
            dimension_semantics=("arbitrary", "arbitrary"),
        ),
    )(hs, rwt, rb, gate_w, gate_b, up_w, up_b, down_proj, down_b)

    return out.reshape(bsz, seq, d), scores


# speedup vs baseline: 1.7765x; 1.7617x over previous
import functools

import jax
import jax.numpy as jnp
from jax.experimental import pallas as pl
from jax.experimental.pallas import tpu as pltpu

E = 8
D = 768
FF = 768
ALPHA = 1.702
LIMIT = 7.0


def _moe_body(hs_ref, rwt_ref, rb_ref, gw_ref, gb_ref, uw_ref, ub_ref,
              dw_ref, db_ref, out_ref, scores_ref, *, bt):
    e = pl.program_id(0)
    t = pl.program_id(1)
    x = hs_ref[...]

    @pl.when(e == 0)
    def _router():
        logits = jnp.dot(x, rwt_ref[...], preferred_element_type=jnp.float32)
        logits = logits + rb_ref[...]
        col = jax.lax.broadcasted_iota(jnp.int32, logits.shape, 1)
        m1 = jnp.max(logits, axis=1, keepdims=True)
        a1 = jnp.min(jnp.where(logits == m1, col, E), axis=1, keepdims=True)
        rest = jnp.where(col == a1, -jnp.inf, logits)
        m2 = jnp.max(rest, axis=1, keepdims=True)
        a2 = jnp.min(jnp.where(rest == m2, col, E), axis=1, keepdims=True)
        p1 = 1.0 / (1.0 + jnp.exp(m2 - m1))
        p2 = 1.0 - p1
        scores = jnp.where(col == a1, p1, jnp.where(col == a2, p2, 0.0))
        scores_ref[pl.ds(t * bt, bt), :] = scores

    scores_blk = scores_ref[pl.ds(t * bt, bt), :]
    col = jax.lax.broadcasted_iota(jnp.int32, scores_blk.shape, 1)
    w = jnp.sum(jnp.where(col == e, scores_blk, 0.0), axis=1, keepdims=True)

    xb = x.astype(jnp.bfloat16)
    g = jnp.dot(xb, gw_ref[0], preferred_element_type=jnp.float32) + gb_ref[0]
    u = jnp.dot(xb, uw_ref[0], preferred_element_type=jnp.float32) + ub_ref[0]
    g = jnp.minimum(g, LIMIT)
    u = jnp.clip(u, -LIMIT, LIMIT)
    glu = g / (1.0 + jnp.exp(-ALPHA * g))
    act = ((u + 1.0) * glu).astype(jnp.bfloat16)
    contrib = jnp.dot(act, dw_ref[0], preferred_element_type=jnp.float32)
    contrib = w * (contrib + db_ref[0])

    sl = pl.ds(t * bt, bt)

    @pl.when(e == 0)
    def _init():
        out_ref[sl, :] = contrib

    @pl.when(e != 0)
    def _acc():
        out_ref[sl, :] = out_ref[sl, :] + contrib


def kernel(hidden_states, router_weight, router_bias, gate_up_proj,
           gate_up_proj_bias, down_proj, down_proj_bias):
    bsz, seq, d = hidden_states.shape
    T = bsz * seq
    hs = hidden_states.reshape(T, d)
    BT = 512
    NT = T // BT

    rwt = router_weight.T
    rb = router_bias.reshape(1, E)
    gate_w = gate_up_proj[..., ::2].astype(jnp.bfloat16)
    up_w = gate_up_proj[..., 1::2].astype(jnp.bfloat16)
    down_w = down_proj.astype(jnp.bfloat16)
    gate_b = gate_up_proj_bias[..., ::2].reshape(E, 1, FF)
    up_b = gate_up_proj_bias[..., 1::2].reshape(E, 1, FF)
    down_b = down_proj_bias.reshape(E, 1, D)

    grid = (E, NT)
    out, scores = pl.pallas_call(
        functools.partial(_moe_body, bt=BT),
        grid=grid,
        in_specs=[
            pl.BlockSpec((BT, D), lambda e, t: (t, 0)),
            pl.BlockSpec((D, E), lambda e, t: (0, 0)),
            pl.BlockSpec((1, E), lambda e, t: (0, 0)),
            pl.BlockSpec((1, D, FF), lambda e, t: (e, 0, 0)),
            pl.BlockSpec((1, 1, FF), lambda e, t: (e, 0, 0)),
            pl.BlockSpec((1, D, FF), lambda e, t: (e, 0, 0)),
            pl.BlockSpec((1, 1, FF), lambda e, t: (e, 0, 0)),
            pl.BlockSpec((1, FF, D), lambda e, t: (e, 0, 0)),
            pl.BlockSpec((1, 1, D), lambda e, t: (e, 0, 0)),
        ],
        out_specs=[
            pl.BlockSpec((T, D), lambda e, t: (0, 0)),
            pl.BlockSpec((T, E), lambda e, t: (0, 0)),
        ],
        out_shape=[
            jax.ShapeDtypeStruct((T, D), jnp.float32),
            jax.ShapeDtypeStruct((T, E), jnp.float32),
        ],
        compiler_params=pltpu.CompilerParams(
            dimension_semantics=("arbitrary", "arbitrary"),
        ),
    )(hs, rwt, rb, gate_w, gate_b, up_w, up_b, down_w, down_b)

    return out.reshape(bsz, seq, d), scores


# speedup vs baseline: 9.1012x; 5.1230x over previous
import functools

import jax
import jax.numpy as jnp
import numpy as np
from jax.experimental import pallas as pl
from jax.experimental.pallas import tpu as pltpu

E = 8
D = 768
FF = 768
ALPHA = 1.702
LIMIT = 7.0

_SEL_NP = np.zeros((2 * FF, FF), dtype=np.float32)
_SEL_NP[::2, :] = np.eye(FF, dtype=np.float32)
_SEL = _SEL_NP.astype(jnp.bfloat16)


def _moe_body(hs_ref, rwt_ref, rb_ref, guw_ref, gub_ref, dw_ref, db_ref,
              sel_ref, out_ref, scores_ref, *, bt):
    e = pl.program_id(0)
    t = pl.program_id(1)
    x = hs_ref[...]

    @pl.when(e == 0)
    def _router():
        logits = jnp.dot(x, rwt_ref[...], preferred_element_type=jnp.float32)
        logits = logits + rb_ref[...]
        col = jax.lax.broadcasted_iota(jnp.int32, logits.shape, 1)
        m1 = jnp.max(logits, axis=1, keepdims=True)
        a1 = jnp.min(jnp.where(logits == m1, col, E), axis=1, keepdims=True)
        rest = jnp.where(col == a1, -jnp.inf, logits)
        m2 = jnp.max(rest, axis=1, keepdims=True)
        a2 = jnp.min(jnp.where(rest == m2, col, E), axis=1, keepdims=True)
        p1 = 1.0 / (1.0 + jnp.exp(m2 - m1))
        p2 = 1.0 - p1
        scores = jnp.where(col == a1, p1, jnp.where(col == a2, p2, 0.0))
        scores_ref[pl.ds(t * bt, bt), :] = scores

    scores_blk = scores_ref[pl.ds(t * bt, bt), :]
    col = jax.lax.broadcasted_iota(jnp.int32, scores_blk.shape, 1)
    w = jnp.sum(jnp.where(col == e, scores_blk, 0.0), axis=1, keepdims=True)

    xb = x.astype(jnp.bfloat16)
    guw = guw_ref[0].astype(jnp.bfloat16)
    gu = jnp.dot(xb, guw, preferred_element_type=jnp.float32) + gub_ref[0]
    gu_r = pltpu.roll(gu, 2 * FF - 1, 1)
    g = jnp.minimum(gu, LIMIT)
    u = jnp.clip(gu_r, -LIMIT, LIMIT)
    glu = g / (1.0 + jnp.exp(-ALPHA * g))
    act2 = ((u + 1.0) * glu).astype(jnp.bfloat16)
    act = jnp.dot(act2, sel_ref[...], preferred_element_type=jnp.float32)
    act = act.astype(jnp.bfloat16)
    dw = dw_ref[0].astype(jnp.bfloat16)
    contrib = jnp.dot(act, dw, preferred_element_type=jnp.float32)
    contrib = w * (contrib + db_ref[0])

    sl = pl.ds(t * bt, bt)

    @pl.when(e == 0)
    def _init():
        out_ref[sl, :] = contrib

    @pl.when(e != 0)
    def _acc():
        out_ref[sl, :] = out_ref[sl, :] + contrib


def kernel(hidden_states, router_weight, router_bias, gate_up_proj,
           gate_up_proj_bias, down_proj, down_proj_bias):
    bsz, seq, d = hidden_states.shape
    T = bsz * seq
    hs = hidden_states.reshape(T, d)
    BT = 512
    NT = T // BT

    rwt = router_weight.T
    rb = router_bias.reshape(1, E)
    gub = gate_up_proj_bias.reshape(E, 1, 2 * FF)
    db = down_proj_bias.reshape(E, 1, D)
    sel = _SEL

    grid = (E, NT)
    out, scores = pl.pallas_call(
        functools.partial(_moe_body, bt=BT),
        grid=grid,
        in_specs=[
            pl.BlockSpec((BT, D), lambda e, t: (t, 0)),
            pl.BlockSpec((D, E), lambda e, t: (0, 0)),
            pl.BlockSpec((1, E), lambda e, t: (0, 0)),
            pl.BlockSpec((1, D, 2 * FF), lambda e, t: (e, 0, 0)),
            pl.BlockSpec((1, 1, 2 * FF), lambda e, t: (e, 0, 0)),
            pl.BlockSpec((1, FF, D), lambda e, t: (e, 0, 0)),
            pl.BlockSpec((1, 1, D), lambda e, t: (e, 0, 0)),
            pl.BlockSpec((2 * FF, FF), lambda e, t: (0, 0)),
        ],
        out_specs=[
            pl.BlockSpec((T, D), lambda e, t: (0, 0)),
            pl.BlockSpec((T, E), lambda e, t: (0, 0)),
        ],
        out_shape=[
            jax.ShapeDtypeStruct((T, D), jnp.float32),
            jax.ShapeDtypeStruct((T, E), jnp.float32),
        ],
        compiler_params=pltpu.CompilerParams(
            dimension_semantics=("arbitrary", "arbitrary"),
        ),
    )(hs, rwt, rb, gate_up_proj, gub, down_proj, db, sel)

    return out.reshape(bsz, seq, d), scores


# speedup vs baseline: 9.8063x; 1.0775x over previous
import functools

import jax
import jax.numpy as jnp
import numpy as np
from jax.experimental import pallas as pl
from jax.experimental.pallas import tpu as pltpu

E = 8
D = 768
FF = 768
ALPHA = 1.702
LIMIT = 7.0

_SEL_NP = np.zeros((2 * FF, FF), dtype=np.float32)
_SEL_NP[::2, :] = np.eye(FF, dtype=np.float32)
_SEL = _SEL_NP.astype(jnp.bfloat16)


def _moe_body(hs_ref, rwt_ref, rb_ref, guw_ref, gub_ref, dw_ref, db_ref,
              sel_ref, out_ref, scores_ref, *, bt):
    e = pl.program_id(0)
    t = pl.program_id(1)
    x = hs_ref[...]

    @pl.when(e == 0)
    def _router():
        logits = jnp.dot(x, rwt_ref[...], preferred_element_type=jnp.float32)
        logits = logits + rb_ref[...]
        col = jax.lax.broadcasted_iota(jnp.int32, logits.shape, 1)
        m1 = jnp.max(logits, axis=1, keepdims=True)
        a1 = jnp.min(jnp.where(logits == m1, col, E), axis=1, keepdims=True)
        rest = jnp.where(col == a1, -jnp.inf, logits)
        m2 = jnp.max(rest, axis=1, keepdims=True)
        a2 = jnp.min(jnp.where(rest == m2, col, E), axis=1, keepdims=True)
        p1 = 1.0 / (1.0 + jnp.exp(m2 - m1))
        p2 = 1.0 - p1
        scores = jnp.where(col == a1, p1, jnp.where(col == a2, p2, 0.0))
        scores_ref[pl.ds(t * bt, bt), :] = scores

    scores_blk = scores_ref[pl.ds(t * bt, bt), :]
    col = jax.lax.broadcasted_iota(jnp.int32, scores_blk.shape, 1)
    w = jnp.sum(jnp.where(col == e, scores_blk, 0.0), axis=1, keepdims=True)

    xb = x.astype(jnp.bfloat16)
    guw = guw_ref[0].astype(jnp.bfloat16)
    gu = jnp.dot(xb, guw, preferred_element_type=jnp.float32) + gub_ref[0]
    gu_r = pltpu.roll(gu, 2 * FF - 1, 1)
    g = jnp.minimum(gu, LIMIT)
    u = jnp.clip(gu_r, -LIMIT, LIMIT)
    glu = g / (1.0 + jnp.exp(-ALPHA * g))
    act2 = ((u + 1.0) * glu).astype(jnp.bfloat16)
    act = jnp.dot(act2, sel_ref[...], preferred_element_type=jnp.float32)
    act = act.astype(jnp.bfloat16)
    dw = dw_ref[0].astype(jnp.bfloat16)
    contrib = jnp.dot(act, dw, preferred_element_type=jnp.float32)
    contrib = w * (contrib + db_ref[0])

    sl = pl.ds(t * bt, bt)

    @pl.when(e == 0)
    def _init():
        out_ref[sl, :] = contrib

    @pl.when(e != 0)
    def _acc():
        out_ref[sl, :] = out_ref[sl, :] + contrib


def kernel(hidden_states, router_weight, router_bias, gate_up_proj,
           gate_up_proj_bias, down_proj, down_proj_bias):
    bsz, seq, d = hidden_states.shape
    T = bsz * seq
    hs = hidden_states.reshape(T, d)
    BT = 1024
    NT = T // BT

    rwt = router_weight.T
    rb = router_bias.reshape(1, E)
    gub = gate_up_proj_bias.reshape(E, 1, 2 * FF)
    db = down_proj_bias.reshape(E, 1, D)
    sel = _SEL

    grid = (E, NT)
    out, scores = pl.pallas_call(
        functools.partial(_moe_body, bt=BT),
        grid=grid,
        in_specs=[
            pl.BlockSpec((BT, D), lambda e, t: (t, 0)),
            pl.BlockSpec((D, E), lambda e, t: (0, 0)),
            pl.BlockSpec((1, E), lambda e, t: (0, 0)),
            pl.BlockSpec((1, D, 2 * FF), lambda e, t: (e, 0, 0)),
            pl.BlockSpec((1, 1, 2 * FF), lambda e, t: (e, 0, 0)),
            pl.BlockSpec((1, FF, D), lambda e, t: (e, 0, 0)),
            pl.BlockSpec((1, 1, D), lambda e, t: (e, 0, 0)),
            pl.BlockSpec((2 * FF, FF), lambda e, t: (0, 0)),
        ],
        out_specs=[
            pl.BlockSpec((T, D), lambda e, t: (0, 0)),
            pl.BlockSpec((T, E), lambda e, t: (0, 0)),
        ],
        out_shape=[
            jax.ShapeDtypeStruct((T, D), jnp.float32),
            jax.ShapeDtypeStruct((T, E), jnp.float32),
        ],
        compiler_params=pltpu.CompilerParams(
            dimension_semantics=("arbitrary", "arbitrary"),
        ),
    )(hs, rwt, rb, gate_up_proj, gub, down_proj, db, sel)

    return out.reshape(bsz, seq, d), scores
